# baseline (speedup 1.0000x reference)
import functools

import jax
import jax.numpy as jnp
from jax import lax
from jax.experimental import pallas as pl
from jax.experimental.pallas import tpu as pltpu
from jax.experimental.pallas import tpu_sc as plsc

_B = 16384
_D = 64
_V = 1000000
_TCOLS = 7812
_VTILE = _TCOLS * 128
_NC = 2
_NS = 16
_NW = _NC * _NS
_BPW = _B // _NW
_CHUNK = 128
_TJOB = 63
_JPG = _TCOLS // _TJOB
_NJOBS = 8 * _JPG // _NW
_ROWS = 8 * _TCOLS * 8

_mesh = plsc.VectorSubcoreMesh(core_axis_name="c", subcore_axis_name="s")


@functools.partial(
    pl.kernel,
    mesh=_mesh,
    compiler_params=pltpu.CompilerParams(use_tc_tiling_on_sc=True),
    out_type=jax.ShapeDtypeStruct((_ROWS, _CHUNK), jnp.float32),
    scratch_types=[
        pltpu.VMEM((2, 8 * _TJOB, _CHUNK), jnp.float32),
        pltpu.SemaphoreType.DMA,
        pltpu.SemaphoreType.DMA,
        pltpu.SemaphoreType.DMA,
        pltpu.SemaphoreType.DMA,
    ],
)
def _depad_kernel(tablet_hbm, tiles_hbm, buf_v, sem0, sem1, wsem0, wsem1):
    wid = lax.axis_index("s") * _NC + lax.axis_index("c")

    def fire(jid, p, sem):
        a = jid // _JPG
        kb = jid % _JPG
        c0 = pl.multiple_of(a * 8, 8)
        r0 = pl.multiple_of(kb * (_TJOB * _CHUNK), _CHUNK)
        for q in range(_TJOB):
            pltpu.async_copy(
                tablet_hbm.at[pl.ds(c0, 8), pl.ds(r0 + q * _CHUNK, _CHUNK)],
                buf_v.at[p, pl.ds(q * 8, 8), :],
                sem,
            )

    def drain(p, sem):
        for q in range(_TJOB):
            pltpu.make_async_copy(
                tablet_hbm.at[pl.ds(0, 8), pl.ds(0, _CHUNK)],
                buf_v.at[p, pl.ds(q * 8, 8), :],
                sem,
            ).wait()

    def write_start(jid, p, wsem):
        a = jid // _JPG
        kb = jid % _JPG
        kk0 = pl.multiple_of((a * _TCOLS + kb * _TJOB) * 8, 8)
        pltpu.async_copy(
            buf_v.at[p], tiles_hbm.at[pl.ds(kk0, 8 * _TJOB), :], wsem)

    def write_wait(p, wsem):
        pltpu.make_async_copy(
            buf_v.at[p], tiles_hbm.at[pl.ds(0, 8 * _TJOB), :], wsem).wait()

    fire(wid, 0, sem0)

    def pair(t, _):
        j0 = wid + _NW * (2 * t)
        j1 = wid + _NW * (2 * t + 1)
        j2 = wid + _NW * (2 * t + 2)

        @pl.when(t > 0)
        def _():
            write_wait(1, wsem1)

        fire(j1, 1, sem1)
        drain(0, sem0)
        write_start(j0, 0, wsem0)
        drain(1, sem1)
        write_start(j1, 1, wsem1)
        write_wait(0, wsem0)
        fire(j2, 0, sem0)
        return 0

    lax.fori_loop(0, (_NJOBS - 1) // 2, pair, 0)
    write_wait(1, wsem1)
    drain(0, sem0)
    write_start(wid + _NW * (_NJOBS - 1), 0, wsem0)
    write_wait(0, wsem0)


@functools.partial(
    pl.kernel,
    mesh=_mesh,
    compiler_params=pltpu.CompilerParams(use_tc_tiling_on_sc=True, needs_layout_passes=False),
    out_type=jax.ShapeDtypeStruct((_D, _B), jnp.float32),
    scratch_types=[
        pltpu.VMEM((_BPW,), jnp.int32),
        pltpu.VMEM((_BPW,), jnp.int32),
        pltpu.VMEM((_D * _BPW // _CHUNK, _CHUNK), jnp.int32),
        pltpu.VMEM((_D, _BPW), jnp.float32),
        pltpu.VMEM(((_V - _VTILE) * _D,), jnp.float32),
        pltpu.SemaphoreType.DMA,
    ],
)
def _gather_kernel(idx_hbm, tflat_hbm, tail_hbm, out_hbm, idx_v, pidx_v,
                   ilist_v, dest_v, tail_v, sem):
    wid = lax.axis_index("s") * _NC + lax.axis_index("c")
    base = wid * _BPW
    pltpu.sync_copy(idx_hbm.at[pl.ds(base, _BPW)], idx_v)
    pltpu.sync_copy(tail_hbm, tail_v)

    for v in range(_BPW // 16):
        civ = jnp.minimum(idx_v[pl.ds(v * 16, 16)], _VTILE - 1)
        pidx_v[pl.ds(v * 16, 16)] = (
            lax.shift_right_logical(civ, 7) * 1024 + lax.bitwise_and(civ, 127))

    def mk(c, _):
        coff = (c // 8) * (_TCOLS * 1024) + (c % 8) * _CHUNK
        for v in range(_BPW // 16):
            ilist_v[c * (_BPW // _CHUNK) + v // 8, pl.ds((v % 8) * 16, 16)] = (
                pidx_v[pl.ds(v * 16, 16)] + coff)
        return 0
    lax.fori_loop(0, _D, mk, 0)

    descs = []
    for j in range(_D * _BPW // _CHUNK):
        descs.append(
            pltpu.async_copy(
                tflat_hbm.at[ilist_v.at[j]],
                dest_v.at[j // 4, pl.ds((j % 4) * _CHUNK, _CHUNK)],
                sem,
            )
        )
    for d in descs:
        d.wait()

    def fix(b, _):
        idxv = idx_v[pl.ds(b * 16, 16)]
        m = idxv >= _VTILE
        cnt = plsc.all_reduce_population_count(m)

        @pl.when(cnt[0] > 0)
        def _():
            toff = jnp.maximum(idxv - _VTILE, 0) * _D

            def fixc(c, _):
                tv = plsc.load_gather(tail_v, [toff + c])
                cur = dest_v[c, pl.ds(b * 16, 16)]
                dest_v[c, pl.ds(b * 16, 16)] = jnp.where(m, tv, cur)
                return 0
            lax.fori_loop(0, _D, fixc, 0)
        return 0
    lax.fori_loop(0, _BPW // 16, fix, 0)

    for a in range(_D // 8):
        pltpu.sync_copy(
            dest_v.at[pl.ds(a * 8, 8), :],
            out_hbm.at[pl.ds(a * 8, 8), pl.ds(base, _BPW)],
        )


def kernel(input, action_embedding):
    idx = input[:, 0].astype(jnp.int32)
    tablet = action_embedding.T
    tailflat = action_embedding[_VTILE:].reshape((_V - _VTILE) * _D)
    tiles = _depad_kernel(tablet)
    tflat = tiles.reshape(_ROWS * _CHUNK)
    outT = _gather_kernel(idx, tflat, tailflat)
    return outT.T


# speedup vs baseline: 1.0049x; 1.0049x over previous
import functools

import jax
import jax.numpy as jnp
from jax import lax
from jax.experimental import pallas as pl
from jax.experimental.pallas import tpu as pltpu
from jax.experimental.pallas import tpu_sc as plsc

_B = 16384
_D = 64
_V = 1000000
_TCOLS = 7812
_VTILE = _TCOLS * 128
_NC = 2
_NS = 16
_NW = _NC * _NS
_BPW = _B // _NW
_CHUNK = 128
_TJOB = 31
_JPG = _TCOLS // _TJOB
_NJOBS = 8 * _JPG // _NW
_ROWS = 8 * _TCOLS * 8

_mesh = plsc.VectorSubcoreMesh(core_axis_name="c", subcore_axis_name="s")


@functools.partial(
    pl.kernel,
    mesh=_mesh,
    compiler_params=pltpu.CompilerParams(use_tc_tiling_on_sc=True),
    out_type=jax.ShapeDtypeStruct((_ROWS, _CHUNK), jnp.float32),
    scratch_types=[
        pltpu.VMEM((2, 8 * _TJOB, _CHUNK), jnp.float32),
        pltpu.SemaphoreType.DMA,
        pltpu.SemaphoreType.DMA,
    ],
)
def _depad_kernel(tablet_hbm, tiles_hbm, buf_v, sem0, sem1):
    wid = lax.axis_index("s") * _NC + lax.axis_index("c")

    def fire(jid, p, sem):
        a = jid // _JPG
        kb = jid % _JPG
        c0 = pl.multiple_of(a * 8, 8)
        r0 = pl.multiple_of(kb * (_TJOB * _CHUNK), _CHUNK)
        for q in range(_TJOB):
            pltpu.async_copy(
                tablet_hbm.at[pl.ds(c0, 8), pl.ds(r0 + q * _CHUNK, _CHUNK)],
                buf_v.at[p, pl.ds(q * 8, 8), :],
                sem,
            )

    def drain(p, sem):
        for q in range(_TJOB):
            pltpu.make_async_copy(
                tablet_hbm.at[pl.ds(0, 8), pl.ds(0, _CHUNK)],
                buf_v.at[p, pl.ds(q * 8, 8), :],
                sem,
            ).wait()

    def write(jid, p):
        a = jid // _JPG
        kb = jid % _JPG
        kk0 = pl.multiple_of((a * _TCOLS + kb * _TJOB) * 8, 8)
        pltpu.sync_copy(buf_v.at[p], tiles_hbm.at[pl.ds(kk0, 8 * _TJOB), :])

    fire(wid, 0, sem0)

    def pair(t, _):
        j0 = wid + _NW * (2 * t)
        j1 = wid + _NW * (2 * t + 1)
        j2 = wid + _NW * (2 * t + 2)
        fire(j1, 1, sem1)
        drain(0, sem0)
        write(j0, 0)
        fire(j2, 0, sem0)
        drain(1, sem1)
        write(j1, 1)
        return 0

    lax.fori_loop(0, (_NJOBS - 1) // 2, pair, 0)
    drain(0, sem0)
    write(wid + _NW * (_NJOBS - 1), 0)


@functools.partial(
    pl.kernel,
    mesh=_mesh,
    compiler_params=pltpu.CompilerParams(use_tc_tiling_on_sc=True, needs_layout_passes=False),
    out_type=jax.ShapeDtypeStruct((_D, _B), jnp.float32),
    scratch_types=[
        pltpu.VMEM((_BPW,), jnp.int32),
        pltpu.VMEM((_BPW,), jnp.int32),
        pltpu.VMEM((_D * _BPW // _CHUNK, _CHUNK), jnp.int32),
        pltpu.VMEM((_D, _BPW), jnp.float32),
        pltpu.VMEM(((_V - _VTILE) * _D,), jnp.float32),
        pltpu.SemaphoreType.DMA,
    ],
)
def _gather_kernel(idx_hbm, tflat_hbm, tail_hbm, out_hbm, idx_v, pidx_v,
                   ilist_v, dest_v, tail_v, sem):
    wid = lax.axis_index("s") * _NC + lax.axis_index("c")
    base = wid * _BPW
    pltpu.sync_copy(idx_hbm.at[pl.ds(base, _BPW)], idx_v)
    pltpu.sync_copy(tail_hbm, tail_v)

    for v in range(_BPW // 16):
        civ = jnp.minimum(idx_v[pl.ds(v * 16, 16)], _VTILE - 1)
        pidx_v[pl.ds(v * 16, 16)] = (
            lax.shift_right_logical(civ, 7) * 1024 + lax.bitwise_and(civ, 127))

    def mk(c, _):
        coff = (c // 8) * (_TCOLS * 1024) + (c % 8) * _CHUNK
        for v in range(_BPW // 16):
            ilist_v[c * (_BPW // _CHUNK) + v // 8, pl.ds((v % 8) * 16, 16)] = (
                pidx_v[pl.ds(v * 16, 16)] + coff)
        return 0
    lax.fori_loop(0, _D, mk, 0)

    descs = []
    for j in range(_D * _BPW // _CHUNK):
        descs.append(
            pltpu.async_copy(
                tflat_hbm.at[ilist_v.at[j]],
                dest_v.at[j // 4, pl.ds((j % 4) * _CHUNK, _CHUNK)],
                sem,
            )
        )
    for d in descs:
        d.wait()

    def fix(b, _):
        idxv = idx_v[pl.ds(b * 16, 16)]
        m = idxv >= _VTILE
        cnt = plsc.all_reduce_population_count(m)

        @pl.when(cnt[0] > 0)
        def _():
            toff = jnp.maximum(idxv - _VTILE, 0) * _D

            def fixc(c, _):
                tv = plsc.load_gather(tail_v, [toff + c])
                cur = dest_v[c, pl.ds(b * 16, 16)]
                dest_v[c, pl.ds(b * 16, 16)] = jnp.where(m, tv, cur)
                return 0
            lax.fori_loop(0, _D, fixc, 0)
        return 0
    lax.fori_loop(0, _BPW // 16, fix, 0)

    for a in range(_D // 8):
        pltpu.sync_copy(
            dest_v.at[pl.ds(a * 8, 8), :],
            out_hbm.at[pl.ds(a * 8, 8), pl.ds(base, _BPW)],
        )


def kernel(input, action_embedding):
    idx = input[:, 0].astype(jnp.int32)
    tablet = action_embedding.T
    tailflat = action_embedding[_VTILE:].reshape((_V - _VTILE) * _D)
    tiles = _depad_kernel(tablet)
    tflat = tiles.reshape(_ROWS * _CHUNK)
    outT = _gather_kernel(idx, tflat, tailflat)
    return outT.T


# speedup vs baseline: 1.0051x; 1.0002x over previous
import functools

import jax
import jax.numpy as jnp
from jax import lax
from jax.experimental import pallas as pl
from jax.experimental.pallas import tpu as pltpu
from jax.experimental.pallas import tpu_sc as plsc

_B = 16384
_D = 64
_V = 1000000
_TCOLS = 7812
_VTILE = _TCOLS * 128
_NC = 2
_NS = 16
_NW = _NC * _NS
_BPW = _B // _NW
_CHUNK = 128
_TJOB = 31
_JPG = _TCOLS // _TJOB
_NJOBS = 8 * _JPG // _NW
_ROWS = 8 * _TCOLS * 8

_mesh = plsc.VectorSubcoreMesh(core_axis_name="c", subcore_axis_name="s")


@functools.partial(
    pl.kernel,
    mesh=_mesh,
    compiler_params=pltpu.CompilerParams(use_tc_tiling_on_sc=True),
    out_type=jax.ShapeDtypeStruct((_ROWS, _CHUNK), jnp.float32),
    scratch_types=[
        pltpu.VMEM((2, 8, _TJOB * _CHUNK), jnp.float32),
        pltpu.VMEM((2, 8 * _TJOB, _CHUNK), jnp.float32),
        pltpu.SemaphoreType.DMA,
        pltpu.SemaphoreType.DMA,
    ],
)
def _depad_kernel(tablet_hbm, tiles_hbm, bufa_v, buf_v, sem0, sem1):
    wid = lax.axis_index("s") * _NC + lax.axis_index("c")

    def fire(jid, p, sem):
        a = jid // _JPG
        kb = jid % _JPG
        c0 = pl.multiple_of(a * 8, 8)
        r0 = pl.multiple_of(kb * (_TJOB * _CHUNK), _CHUNK)
        pltpu.async_copy(
            tablet_hbm.at[pl.ds(c0, 8), pl.ds(r0, _TJOB * _CHUNK)],
            bufa_v.at[p],
            sem,
        )

    def drain(p, sem):
        pltpu.make_async_copy(
            tablet_hbm.at[pl.ds(0, 8), pl.ds(0, _TJOB * _CHUNK)],
            bufa_v.at[p],
            sem,
        ).wait()

    def shuffle(p):
        def perq(q, _):
            for s in range(8):
                for k in range(_CHUNK // 16):
                    buf_v[p, q * 8 + s, pl.ds(k * 16, 16)] = (
                        bufa_v[p, s, pl.ds(q * _CHUNK + k * 16, 16)])
            return 0
        lax.fori_loop(0, _TJOB, perq, 0)

    def write(jid, p):
        a = jid // _JPG
        kb = jid % _JPG
        kk0 = pl.multiple_of((a * _TCOLS + kb * _TJOB) * 8, 8)
        pltpu.sync_copy(buf_v.at[p], tiles_hbm.at[pl.ds(kk0, 8 * _TJOB), :])

    fire(wid, 0, sem0)

    def pair(t, _):
        j0 = wid + _NW * (2 * t)
        j1 = wid + _NW * (2 * t + 1)
        j2 = wid + _NW * (2 * t + 2)
        fire(j1, 1, sem1)
        drain(0, sem0)
        shuffle(0)
        fire(j2, 0, sem0)
        write(j0, 0)
        drain(1, sem1)
        shuffle(1)
        write(j1, 1)
        return 0

    lax.fori_loop(0, (_NJOBS - 1) // 2, pair, 0)
    drain(0, sem0)
    shuffle(0)
    write(wid + _NW * (_NJOBS - 1), 0)


@functools.partial(
    pl.kernel,
    mesh=_mesh,
    compiler_params=pltpu.CompilerParams(use_tc_tiling_on_sc=True, needs_layout_passes=False),
    out_type=jax.ShapeDtypeStruct((_D, _B), jnp.float32),
    scratch_types=[
        pltpu.VMEM((_BPW,), jnp.int32),
        pltpu.VMEM((_BPW,), jnp.int32),
        pltpu.VMEM((_D * _BPW // _CHUNK, _CHUNK), jnp.int32),
        pltpu.VMEM((_D, _BPW), jnp.float32),
        pltpu.VMEM(((_V - _VTILE) * _D,), jnp.float32),
        pltpu.SemaphoreType.DMA,
    ],
)
def _gather_kernel(idx_hbm, tflat_hbm, tail_hbm, out_hbm, idx_v, pidx_v,
                   ilist_v, dest_v, tail_v, sem):
    wid = lax.axis_index("s") * _NC + lax.axis_index("c")
    base = wid * _BPW
    pltpu.sync_copy(idx_hbm.at[pl.ds(base, _BPW)], idx_v)
    pltpu.sync_copy(tail_hbm, tail_v)

    for v in range(_BPW // 16):
        civ = jnp.minimum(idx_v[pl.ds(v * 16, 16)], _VTILE - 1)
        pidx_v[pl.ds(v * 16, 16)] = (
            lax.shift_right_logical(civ, 7) * 1024 + lax.bitwise_and(civ, 127))

    def mk(c, _):
        coff = (c // 8) * (_TCOLS * 1024) + (c % 8) * _CHUNK
        for v in range(_BPW // 16):
            ilist_v[c * (_BPW // _CHUNK) + v // 8, pl.ds((v % 8) * 16, 16)] = (
                pidx_v[pl.ds(v * 16, 16)] + coff)
        return 0
    lax.fori_loop(0, _D, mk, 0)

    descs = []
    for j in range(_D * _BPW // _CHUNK):
        descs.append(
            pltpu.async_copy(
                tflat_hbm.at[ilist_v.at[j]],
                dest_v.at[j // 4, pl.ds((j % 4) * _CHUNK, _CHUNK)],
                sem,
            )
        )
    for d in descs:
        d.wait()

    def fix(b, _):
        idxv = idx_v[pl.ds(b * 16, 16)]
        m = idxv >= _VTILE
        cnt = plsc.all_reduce_population_count(m)

        @pl.when(cnt[0] > 0)
        def _():
            toff = jnp.maximum(idxv - _VTILE, 0) * _D

            def fixc(c, _):
                tv = plsc.load_gather(tail_v, [toff + c])
                cur = dest_v[c, pl.ds(b * 16, 16)]
                dest_v[c, pl.ds(b * 16, 16)] = jnp.where(m, tv, cur)
                return 0
            lax.fori_loop(0, _D, fixc, 0)
        return 0
    lax.fori_loop(0, _BPW // 16, fix, 0)

    for a in range(_D // 8):
        pltpu.sync_copy(
            dest_v.at[pl.ds(a * 8, 8), :],
            out_hbm.at[pl.ds(a * 8, 8), pl.ds(base, _BPW)],
        )


def kernel(input, action_embedding):
    idx = input[:, 0].astype(jnp.int32)
    tablet = action_embedding.T
    tailflat = action_embedding[_VTILE:].reshape((_V - _VTILE) * _D)
    tiles = _depad_kernel(tablet)
    tflat = tiles.reshape(_ROWS * _CHUNK)
    outT = _gather_kernel(idx, tflat, tailflat)
    return outT.T


# speedup vs baseline: 1.0072x; 1.0021x over previous
import functools

import jax
import jax.numpy as jnp
from jax import lax
from jax.experimental import pallas as pl
from jax.experimental.pallas import tpu as pltpu
from jax.experimental.pallas import tpu_sc as plsc

_B = 16384
_D = 64
_V = 1000000
_TCOLS = 7812
_VTILE = _TCOLS * 128
_NC = 2
_NS = 16
_NW = _NC * _NS
_BPW = _B // _NW
_CHUNK = 128
_TJOB = 31
_JPG = _TCOLS // _TJOB
_NJOBS = 8 * _JPG // _NW
_ROWS = 8 * _TCOLS * 8

_mesh = plsc.VectorSubcoreMesh(core_axis_name="c", subcore_axis_name="s")


@functools.partial(
    pl.kernel,
    mesh=_mesh,
    compiler_params=pltpu.CompilerParams(use_tc_tiling_on_sc=True),
    out_type=jax.ShapeDtypeStruct((_ROWS, _CHUNK), jnp.float32),
    scratch_types=[
        pltpu.VMEM((2, 8 * _TJOB, _CHUNK), jnp.float32),
        pltpu.SemaphoreType.DMA,
        pltpu.SemaphoreType.DMA,
    ],
)
def _depad_kernel(tablet_hbm, tiles_hbm, buf_v, sem0, sem1):
    wid = lax.axis_index("s") * _NC + lax.axis_index("c")

    def fire(jid, p, sem):
        a = jid // _JPG
        kb = jid % _JPG
        c0 = pl.multiple_of(a * 8, 8)
        r0 = pl.multiple_of(kb * (_TJOB * _CHUNK), _CHUNK)
        for q in range(_TJOB):
            pltpu.async_copy(
                tablet_hbm.at[pl.ds(c0, 8), pl.ds(r0 + q * _CHUNK, _CHUNK)],
                buf_v.at[p, pl.ds(q * 8, 8), :],
                sem,
            )

    def drain(p, sem):
        for q in range(_TJOB):
            pltpu.make_async_copy(
                tablet_hbm.at[pl.ds(0, 8), pl.ds(0, _CHUNK)],
                buf_v.at[p, pl.ds(q * 8, 8), :],
                sem,
            ).wait()

    def write(jid, p):
        a = jid // _JPG
        kb = jid % _JPG
        kk0 = pl.multiple_of((a * _TCOLS + kb * _TJOB) * 8, 8)
        pltpu.sync_copy(buf_v.at[p], tiles_hbm.at[pl.ds(kk0, 8 * _TJOB), :])

    fire(wid, 0, sem0)

    def pair(t, _):
        j0 = wid + _NW * (2 * t)
        j1 = wid + _NW * (2 * t + 1)
        j2 = wid + _NW * (2 * t + 2)
        fire(j1, 1, sem1)
        drain(0, sem0)
        write(j0, 0)
        fire(j2, 0, sem0)
        drain(1, sem1)
        write(j1, 1)
        return 0

    lax.fori_loop(0, (_NJOBS - 1) // 2, pair, 0)
    drain(0, sem0)
    write(wid + _NW * (_NJOBS - 1), 0)


@functools.partial(
    pl.kernel,
    mesh=_mesh,
    compiler_params=pltpu.CompilerParams(use_tc_tiling_on_sc=True, needs_layout_passes=False),
    out_type=jax.ShapeDtypeStruct((_D, _B), jnp.float32),
    scratch_types=[
        pltpu.VMEM((_BPW,), jnp.int32),
        pltpu.VMEM((_BPW,), jnp.int32),
        pltpu.VMEM((_D * _BPW // _CHUNK, _CHUNK), jnp.int32),
        pltpu.VMEM((_D, _BPW), jnp.float32),
        pltpu.VMEM(((_V - _VTILE) * _D,), jnp.float32),
        pltpu.SemaphoreType.DMA,
    ],
)
def _gather_kernel(idx_hbm, tflat_hbm, tail_hbm, out_hbm, idx_v, pidx_v,
                   ilist_v, dest_v, tail_v, sem):
    wid = lax.axis_index("s") * _NC + lax.axis_index("c")
    base = wid * _BPW
    pltpu.sync_copy(idx_hbm.at[pl.ds(base, _BPW)], idx_v)
    pltpu.sync_copy(tail_hbm, tail_v)

    for v in range(_BPW // 16):
        civ = jnp.minimum(idx_v[pl.ds(v * 16, 16)], _VTILE - 1)
        pidx_v[pl.ds(v * 16, 16)] = (
            lax.shift_right_logical(civ, 7) * 1024 + lax.bitwise_and(civ, 127))

    def mk(c, _):
        coff = (c // 8) * (_TCOLS * 1024) + (c % 8) * _CHUNK
        for v in range(_BPW // 16):
            ilist_v[c * (_BPW // _CHUNK) + v // 8, pl.ds((v % 8) * 16, 16)] = (
                pidx_v[pl.ds(v * 16, 16)] + coff)
        return 0
    lax.fori_loop(0, _D, mk, 0)

    descs = []
    for j in range(_D * _BPW // _CHUNK):
        descs.append(
            pltpu.async_copy(
                tflat_hbm.at[ilist_v.at[j]],
                dest_v.at[j // 4, pl.ds((j % 4) * _CHUNK, _CHUNK)],
                sem,
            )
        )
    for d in descs:
        d.wait()

    def fix(b, _):
        idxv = idx_v[pl.ds(b * 16, 16)]
        m = idxv >= _VTILE
        cnt = plsc.all_reduce_population_count(m)

        @pl.when(cnt[0] > 0)
        def _():
            toff = jnp.maximum(idxv - _VTILE, 0) * _D

            def fixc(c, _):
                tv = plsc.load_gather(tail_v, [toff + c])
                cur = dest_v[c, pl.ds(b * 16, 16)]
                dest_v[c, pl.ds(b * 16, 16)] = jnp.where(m, tv, cur)
                return 0
            lax.fori_loop(0, _D, fixc, 0)
        return 0
    lax.fori_loop(0, _BPW // 16, fix, 0)

    for a in range(_D // 8):
        pltpu.sync_copy(
            dest_v.at[pl.ds(a * 8, 8), :],
            out_hbm.at[pl.ds(a * 8, 8), pl.ds(base, _BPW)],
        )


def kernel(input, action_embedding):
    idx = input[:, 0].astype(jnp.int32)
    tablet = action_embedding.T
    tailflat = action_embedding[_VTILE:].reshape((_V - _VTILE) * _D)
    tiles = _depad_kernel(tablet)
    tflat = tiles.reshape(_ROWS * _CHUNK)
    outT = _gather_kernel(idx, tflat, tailflat)
    return outT.T
